# Initial kernel scaffold; baseline (speedup 1.0000x reference)
import jax
import jax.numpy as jnp
from jax.experimental import pallas as pl


def kernel(x, gain, bias, log_mix, log_scale):
    raise NotImplementedError("write your pallas kernel here")


# speedup vs baseline: 15.1407x; 15.1407x over previous
import functools
import math

import jax
import jax.numpy as jnp
from jax.experimental import pallas as pl
from jax.experimental.pallas import tpu as pltpu

K_NEIGHBORS = 8


def _dgn_kernel(params_ref, q_ref, k_ref, gain_ref, bias_ref, o_ref, *, bq, t):
    i = pl.program_id(1)
    q = q_ref[0]
    keys = k_ref[0]

    scores = jax.lax.dot_general(
        q, keys, (((1,), (1,)), ((), ())),
        preferred_element_type=jnp.float32)

    neg = jnp.finfo(jnp.float32).min
    rows = i * bq + jax.lax.broadcasted_iota(jnp.int32, (bq, t), 0)
    cols = jax.lax.broadcasted_iota(jnp.int32, (bq, t), 1)
    scores = jnp.where(cols <= rows, scores, neg)

    adj = jnp.zeros((bq, t), jnp.float32)
    deg = jnp.zeros((bq, 1), jnp.float32)
    for _ in range(K_NEIGHBORS):
        m = jnp.max(scores, axis=1, keepdims=True)
        is_max = scores == m
        amin = jnp.min(jnp.where(is_max, cols, t), axis=1, keepdims=True)
        valid = m > neg / 2
        hit = cols == amin
        adj = jnp.where(jnp.logical_and(hit, valid), 1.0, adj)
        deg += valid.astype(jnp.float32)
        scores = jnp.where(hit, neg, scores)

    msg = jax.lax.dot_general(
        adj, keys, (((1,), (0,)), ((), ())),
        preferred_element_type=jnp.float32)
    msg = msg / jnp.maximum(deg, 1.0)

    mix = params_ref[0]
    scale = params_ref[1]
    blended = mix * q + (1.0 - mix) * msg
    z = blended * gain_ref[...] + bias_ref[...]
    delta = 0.5 * z * (1.0 + jax.lax.erf(z / math.sqrt(2.0))) * scale
    o_ref[0] = delta


@jax.jit
def kernel(x, gain, bias, log_mix, log_scale):
    b, t, d = x.shape
    bq = 256
    mix = jax.nn.sigmoid(log_mix)
    scale = jax.nn.softplus(log_scale) + 0.01
    params = jnp.stack([mix, scale]).astype(jnp.float32)

    grid = (b, t // bq)
    out = pl.pallas_call(
        functools.partial(_dgn_kernel, bq=bq, t=t),
        grid=grid,
        in_specs=[
            pl.BlockSpec(memory_space=pltpu.SMEM),
            pl.BlockSpec((1, bq, d), lambda bi, qi: (bi, qi, 0)),
            pl.BlockSpec((1, t, d), lambda bi, qi: (bi, 0, 0)),
            pl.BlockSpec((d,), lambda bi, qi: (0,)),
            pl.BlockSpec((d,), lambda bi, qi: (0,)),
        ],
        out_specs=pl.BlockSpec((1, bq, d), lambda bi, qi: (bi, qi, 0)),
        out_shape=jax.ShapeDtypeStruct((b, t, d), jnp.float32),
    )(params, x, x, gain, bias)
    return out
